# baseline (speedup 1.0000x reference)
import functools

import jax
import jax.numpy as jnp
from jax import lax
from jax.experimental import pallas as pl
from jax.experimental.pallas import tpu as pltpu
from jax.experimental.pallas import tpu_sc as plsc

_EMBED_DIM = 64
_SEQ_LEN = 200
_K = 10000.0

_NUM_WORKERS = 32
_ROWS_TOTAL = 4096 * 200
_ROWS_PER_W = _ROWS_TOTAL // _NUM_WORKERS
_SEQS_PER_W = _ROWS_PER_W // _SEQ_LEN
_GATHER = 100
_GATHERS_PER_SEQ = _SEQ_LEN // _GATHER


def _enc_body(o_ref):
    pos = lax.broadcasted_iota(jnp.int32, (_SEQ_LEN, _EMBED_DIM), 0).astype(
        jnp.float32)
    j = lax.broadcasted_iota(jnp.int32, (_SEQ_LEN, _EMBED_DIM), 1)
    i = (j // 2).astype(jnp.float32)
    denom = jnp.exp(i * (2.0 / _EMBED_DIM) * jnp.log(_K))
    ang = pos / denom
    o_ref[...] = jnp.where(j % 2 == 0, jnp.sin(ang), jnp.cos(ang))


def _make_enc():
    return pl.pallas_call(
        _enc_body,
        out_shape=jax.ShapeDtypeStruct((_SEQ_LEN, _EMBED_DIM), jnp.float32),
    )()


_mesh = plsc.VectorSubcoreMesh(core_axis_name="c", subcore_axis_name="s")


@functools.partial(
    pl.kernel,
    mesh=_mesh,
    out_type=jax.ShapeDtypeStruct((_ROWS_TOTAL, _EMBED_DIM), jnp.float32),
    scratch_types=[
        pltpu.VMEM((_ROWS_PER_W // _GATHER, _GATHER), jnp.int32),
        pltpu.VMEM((_SEQ_LEN * _EMBED_DIM,), jnp.float32),
        pltpu.VMEM((_SEQ_LEN, _EMBED_DIM), jnp.float32),
        pltpu.SemaphoreType.DMA,
    ],
    compiler_params=pltpu.CompilerParams(use_tc_tiling_on_sc=False),
)
def _sc_gather_add(w_hbm, idx_hbm, enc_hbm, out_hbm, idx_v, enc_v, rows_v, sem):
    nc = 2
    wid = lax.axis_index("s") * nc + lax.axis_index("c")
    row_base = wid * _ROWS_PER_W
    g_base = wid * (_ROWS_PER_W // _GATHER)

    pltpu.sync_copy(idx_hbm.at[pl.ds(g_base, _ROWS_PER_W // _GATHER), :], idx_v)
    pltpu.sync_copy(enc_hbm, enc_v)

    def add_row(l, _):
        for d in range(_EMBED_DIM // 16):
            e = enc_v[pl.ds(l * _EMBED_DIM + d * 16, 16)]
            plsc.addupdate(rows_v.at[l, pl.ds(d * 16, 16)], e)
        return 0

    def seq_body(c, _):
        for j in range(_GATHERS_PER_SEQ):
            pltpu.async_copy(
                w_hbm.at[idx_v.at[c * _GATHERS_PER_SEQ + j]],
                rows_v.at[pl.ds(j * _GATHER, _GATHER), :],
                sem,
            ).wait()
        lax.fori_loop(0, _SEQ_LEN, add_row, 0)
        pltpu.sync_copy(
            rows_v, out_hbm.at[pl.ds(row_base + c * _SEQ_LEN, _SEQ_LEN), :]
        )
        return 0

    lax.fori_loop(0, _SEQS_PER_W, seq_body, 0)


def kernel(x, W):
    b, l = x.shape
    xf = x.reshape(_ROWS_TOTAL // _GATHER, _GATHER)
    enc = _make_enc().reshape(_SEQ_LEN * _EMBED_DIM)
    out = _sc_gather_add(W, xf, enc)
    return out.reshape(b, l, _EMBED_DIM)


# speedup vs baseline: 1.0430x; 1.0430x over previous
import functools

import jax
import jax.numpy as jnp
from jax import lax
from jax.experimental import pallas as pl
from jax.experimental.pallas import tpu as pltpu
from jax.experimental.pallas import tpu_sc as plsc

_EMBED_DIM = 64
_SEQ_LEN = 200
_K = 10000.0

_NUM_WORKERS = 32
_ROWS_TOTAL = 4096 * 200
_ROWS_PER_W = _ROWS_TOTAL // _NUM_WORKERS
_SEQS_PER_W = _ROWS_PER_W // _SEQ_LEN
_GATHER = 100
_GATHERS_PER_SEQ = _SEQ_LEN // _GATHER


def _enc_body(o_ref):
    pos = lax.broadcasted_iota(jnp.int32, (_SEQ_LEN, _EMBED_DIM), 0).astype(
        jnp.float32)
    j = lax.broadcasted_iota(jnp.int32, (_SEQ_LEN, _EMBED_DIM), 1)
    i = (j // 2).astype(jnp.float32)
    denom = jnp.exp(i * (2.0 / _EMBED_DIM) * jnp.log(_K))
    ang = pos / denom
    o_ref[...] = jnp.where(j % 2 == 0, jnp.sin(ang), jnp.cos(ang))


def _make_enc():
    return pl.pallas_call(
        _enc_body,
        out_shape=jax.ShapeDtypeStruct((_SEQ_LEN, _EMBED_DIM), jnp.float32),
    )()


_mesh = plsc.VectorSubcoreMesh(core_axis_name="c", subcore_axis_name="s")


_NBUF = 4


@functools.partial(
    pl.kernel,
    mesh=_mesh,
    out_type=jax.ShapeDtypeStruct((_ROWS_TOTAL, _EMBED_DIM), jnp.float32),
    scratch_types=[
        pltpu.VMEM((_ROWS_PER_W // _GATHER, _GATHER), jnp.int32),
        pltpu.VMEM((_SEQ_LEN * _EMBED_DIM,), jnp.float32),
        pltpu.VMEM((_NBUF, _SEQ_LEN, _EMBED_DIM), jnp.float32),
        pltpu.SemaphoreType.DMA((_NBUF,)),
        pltpu.SemaphoreType.DMA((_NBUF,)),
    ],
    compiler_params=pltpu.CompilerParams(use_tc_tiling_on_sc=False),
)
def _sc_gather_add(w_hbm, idx_hbm, enc_hbm, out_hbm,
                   idx_v, enc_v, rows_v, gsem, osem):
    nc = 2
    wid = lax.axis_index("s") * nc + lax.axis_index("c")
    row_base = wid * _ROWS_PER_W
    g_base = wid * (_ROWS_PER_W // _GATHER)

    pltpu.sync_copy(idx_hbm.at[pl.ds(g_base, _ROWS_PER_W // _GATHER), :], idx_v)
    pltpu.sync_copy(enc_hbm, enc_v)

    def fire_gather(c, b):
        for j in range(_GATHERS_PER_SEQ):
            pltpu.async_copy(
                w_hbm.at[idx_v.at[c * _GATHERS_PER_SEQ + j]],
                rows_v.at[b, pl.ds(j * _GATHER, _GATHER), :],
                gsem.at[b],
            )

    def wait_gather(b):
        for j in range(_GATHERS_PER_SEQ):
            pltpu.make_async_copy(
                w_hbm.at[idx_v.at[j]],
                rows_v.at[b, pl.ds(j * _GATHER, _GATHER), :],
                gsem.at[b],
            ).wait()

    def fire_out(c, b):
        pltpu.async_copy(
            rows_v.at[b],
            out_hbm.at[pl.ds(row_base + c * _SEQ_LEN, _SEQ_LEN), :],
            osem.at[b],
        )

    def wait_out(b):
        pltpu.make_async_copy(
            rows_v.at[b],
            out_hbm.at[pl.ds(row_base, _SEQ_LEN), :],
            osem.at[b],
        ).wait()

    for b in range(_NBUF - 1):
        fire_gather(b, b)

    def add_row(b):
        def body(l, _):
            for d in range(_EMBED_DIM // 16):
                e = enc_v[pl.ds(l * _EMBED_DIM + d * 16, 16)]
                plsc.addupdate(rows_v.at[b, l, pl.ds(d * 16, 16)], e)
            return 0
        lax.fori_loop(0, _SEQ_LEN, body, 0, unroll=4)

    def seq_body(c, _):
        b = lax.rem(c, _NBUF)
        wait_gather(b)
        add_row(b)
        fire_out(c, b)
        bp = lax.rem(c + _NBUF - 1, _NBUF)

        @pl.when(c >= 1)
        def _():
            wait_out(bp)

        @pl.when(c + _NBUF - 1 < _SEQS_PER_W)
        def _():
            fire_gather(c + _NBUF - 1, bp)

        return 0

    lax.fori_loop(0, _SEQS_PER_W, seq_body, 0)
    wait_out(lax.rem(jnp.int32(_SEQS_PER_W - 1), _NBUF))


def kernel(x, W):
    b, l = x.shape
    xf = x.reshape(_ROWS_TOTAL // _GATHER, _GATHER)
    enc = _make_enc().reshape(_SEQ_LEN * _EMBED_DIM)
    out = _sc_gather_add(W, xf, enc)
    return out.reshape(b, l, _EMBED_DIM)


# speedup vs baseline: 1.2034x; 1.1538x over previous
import functools

import jax
import jax.numpy as jnp
from jax import lax
from jax.experimental import pallas as pl
from jax.experimental.pallas import tpu as pltpu
from jax.experimental.pallas import tpu_sc as plsc

_EMBED_DIM = 64
_SEQ_LEN = 200
_K = 10000.0

_NUM_ROWS = 1000000
_BATCH = 4096
_NUM_WORKERS = 32
_ROWS_TOTAL = _BATCH * _SEQ_LEN
_ROWS_PER_W = _ROWS_TOTAL // _NUM_WORKERS
_CHUNK = 40
_CHUNKS_PER_SEQ = _SEQ_LEN // _CHUNK
_CHUNKS_PER_W = _ROWS_PER_W // _CHUNK
_NG = 8
_NO = 3


def _enc_body(o_ref):
    pos = lax.broadcasted_iota(jnp.int32, (_SEQ_LEN, _EMBED_DIM), 0).astype(
        jnp.float32)
    j = lax.broadcasted_iota(jnp.int32, (_SEQ_LEN, _EMBED_DIM), 1)
    i = (j // 2).astype(jnp.float32)
    denom = jnp.exp(i * (2.0 / _EMBED_DIM) * jnp.log(_K))
    ang = pos / denom
    o_ref[...] = jnp.where(j % 2 == 0, jnp.sin(ang), jnp.cos(ang))


def _make_enc():
    return pl.pallas_call(
        _enc_body,
        out_shape=jax.ShapeDtypeStruct((_SEQ_LEN, _EMBED_DIM), jnp.float32),
    )()


_mesh = plsc.VectorSubcoreMesh(core_axis_name="c", subcore_axis_name="s")


@functools.partial(
    pl.kernel,
    mesh=_mesh,
    out_type=jax.ShapeDtypeStruct((_ROWS_TOTAL, _EMBED_DIM), jnp.float32),
    scratch_types=[
        pltpu.VMEM((_ROWS_PER_W,), jnp.int32),
        pltpu.VMEM((_SEQ_LEN * _EMBED_DIM,), jnp.float32),
        pltpu.VMEM((_NG, _CHUNK, 128), jnp.float32),
        pltpu.VMEM((_NO, _CHUNK, _EMBED_DIM), jnp.float32),
        pltpu.SemaphoreType.DMA((_NG,)),
        pltpu.SemaphoreType.DMA((_NO,)),
    ],
)
def _sc_gather_add(w_hbm, idx_hbm, enc_hbm, out_hbm,
                   idx_v, enc_v, rows_v, obuf_v, gsem, osem):
    nc = 2
    wid = lax.axis_index("s") * nc + lax.axis_index("c")
    row_base = wid * _ROWS_PER_W

    pltpu.sync_copy(idx_hbm.at[pl.ds(row_base, _ROWS_PER_W)], idx_v)
    pltpu.sync_copy(enc_hbm, enc_v)

    def fire_gather(c, bg):
        pltpu.async_copy(
            w_hbm.at[idx_v.at[pl.ds(c * _CHUNK, _CHUNK)]],
            rows_v.at[bg],
            gsem.at[bg],
        )

    def wait_gather(bg):
        pltpu.make_async_copy(
            w_hbm.at[idx_v.at[pl.ds(0, _CHUNK)]],
            rows_v.at[bg],
            gsem.at[bg],
        ).wait()

    def fire_out(c, bo):
        pltpu.async_copy(
            obuf_v.at[bo],
            out_hbm.at[pl.ds(row_base + c * _CHUNK, _CHUNK), :],
            osem.at[bo],
        )

    def wait_out(bo):
        pltpu.make_async_copy(
            obuf_v.at[bo],
            out_hbm.at[pl.ds(row_base, _CHUNK), :],
            osem.at[bo],
        ).wait()

    for b in range(_NG):
        fire_gather(b, b)

    def add_chunk(bg, bo, eoff):
        def body(l, _):
            for d in range(_EMBED_DIM // 16):
                e = enc_v[pl.ds(eoff + l * _EMBED_DIM + d * 16, 16)]
                r = rows_v[bg, l, pl.ds(d * 16, 16)]
                obuf_v[bo, l, pl.ds(d * 16, 16)] = r + e
            return 0
        lax.fori_loop(0, _CHUNK, body, 0, unroll=4)

    def chunk_body(c, _):
        bg = lax.rem(c, _NG)
        bo = lax.rem(c, _NO)
        eoff = lax.rem(c, _CHUNKS_PER_SEQ) * (_CHUNK * _EMBED_DIM)

        @pl.when(c >= _NO)
        def _():
            wait_out(bo)

        wait_gather(bg)
        add_chunk(bg, bo, eoff)
        fire_out(c, bo)

        @pl.when(c + _NG < _CHUNKS_PER_W)
        def _():
            fire_gather(c + _NG, bg)

        return 0

    lax.fori_loop(0, _CHUNKS_PER_W, chunk_body, 0)
    for b in range(_NO):
        wait_out(b)


def kernel(x, W):
    wp = jnp.pad(W, ((0, 0), (0, 128 - _EMBED_DIM)))
    xf = x.reshape(_ROWS_TOTAL)
    enc = _make_enc().reshape(_SEQ_LEN * _EMBED_DIM)
    out = _sc_gather_add(wp, xf, enc)
    return out.reshape(_BATCH, _SEQ_LEN, _EMBED_DIM)


# speedup vs baseline: 1.2072x; 1.0031x over previous
import functools

import jax
import jax.numpy as jnp
from jax import lax
from jax.experimental import pallas as pl
from jax.experimental.pallas import tpu as pltpu
from jax.experimental.pallas import tpu_sc as plsc

_EMBED_DIM = 64
_SEQ_LEN = 200
_K = 10000.0

_NUM_ROWS = 1000000
_BATCH = 4096
_NUM_WORKERS = 32
_ROWS_TOTAL = _BATCH * _SEQ_LEN
_ROWS_PER_W = _ROWS_TOTAL // _NUM_WORKERS
_CHUNK = 80
_CHUNKS_PER_W = _ROWS_PER_W // _CHUNK
_NG = 5
_NO = 3


def _enc_body(o_ref):
    pos = lax.broadcasted_iota(jnp.int32, (_SEQ_LEN, _EMBED_DIM), 0).astype(
        jnp.float32)
    j = lax.broadcasted_iota(jnp.int32, (_SEQ_LEN, _EMBED_DIM), 1)
    i = (j // 2).astype(jnp.float32)
    denom = jnp.exp(i * (2.0 / _EMBED_DIM) * jnp.log(_K))
    ang = pos / denom
    o_ref[...] = jnp.where(j % 2 == 0, jnp.sin(ang), jnp.cos(ang))


def _make_enc():
    return pl.pallas_call(
        _enc_body,
        out_shape=jax.ShapeDtypeStruct((_SEQ_LEN, _EMBED_DIM), jnp.float32),
    )()


_mesh = plsc.VectorSubcoreMesh(core_axis_name="c", subcore_axis_name="s")


@functools.partial(
    pl.kernel,
    mesh=_mesh,
    out_type=jax.ShapeDtypeStruct((_ROWS_TOTAL, _EMBED_DIM), jnp.float32),
    scratch_types=[
        pltpu.VMEM((_ROWS_PER_W,), jnp.int32),
        pltpu.VMEM((_SEQ_LEN * _EMBED_DIM,), jnp.float32),
        pltpu.VMEM((_NG, _CHUNK, 128), jnp.float32),
        pltpu.VMEM((_NO, _CHUNK, _EMBED_DIM), jnp.float32),
        pltpu.SemaphoreType.DMA((_NG,)),
        pltpu.SemaphoreType.DMA((_NO,)),
    ],
)
def _sc_gather_add(w_hbm, idx_hbm, enc_hbm, out_hbm,
                   idx_v, enc_v, rows_v, obuf_v, gsem, osem):
    nc = 2
    wid = lax.axis_index("s") * nc + lax.axis_index("c")
    row_base = wid * _ROWS_PER_W

    pltpu.sync_copy(idx_hbm.at[pl.ds(row_base, _ROWS_PER_W)], idx_v)
    pltpu.sync_copy(enc_hbm, enc_v)

    def fire_gather(c, bg):
        pltpu.async_copy(
            w_hbm.at[idx_v.at[pl.ds(c * _CHUNK, _CHUNK)]],
            rows_v.at[bg],
            gsem.at[bg],
        )

    def wait_gather(bg):
        pltpu.make_async_copy(
            w_hbm.at[idx_v.at[pl.ds(0, _CHUNK)]],
            rows_v.at[bg],
            gsem.at[bg],
        ).wait()

    def fire_out(c, bo):
        pltpu.async_copy(
            obuf_v.at[bo],
            out_hbm.at[pl.ds(row_base + c * _CHUNK, _CHUNK), :],
            osem.at[bo],
        )

    def wait_out(bo):
        pltpu.make_async_copy(
            obuf_v.at[bo],
            out_hbm.at[pl.ds(row_base, _CHUNK), :],
            osem.at[bo],
        ).wait()

    for b in range(_NG):
        fire_gather(b, b)

    def add_chunk(bg, bo, eoff):
        def body(l, _):
            el = lax.rem(eoff + l, _SEQ_LEN) * _EMBED_DIM
            for d in range(_EMBED_DIM // 16):
                e = enc_v[pl.ds(el + d * 16, 16)]
                r = rows_v[bg, l, pl.ds(d * 16, 16)]
                obuf_v[bo, l, pl.ds(d * 16, 16)] = r + e
            return 0
        lax.fori_loop(0, _CHUNK, body, 0, unroll=4)

    def chunk_body(c, _):
        bg = lax.rem(c, _NG)
        bo = lax.rem(c, _NO)
        eoff = lax.rem(c * _CHUNK, _SEQ_LEN)

        @pl.when(c >= _NO)
        def _():
            wait_out(bo)

        wait_gather(bg)
        add_chunk(bg, bo, eoff)
        fire_out(c, bo)

        @pl.when(c + _NG < _CHUNKS_PER_W)
        def _():
            fire_gather(c + _NG, bg)

        return 0

    lax.fori_loop(0, _CHUNKS_PER_W, chunk_body, 0)
    for b in range(_NO):
        wait_out(b)


def kernel(x, W):
    wp = jnp.pad(W, ((0, 0), (0, 128 - _EMBED_DIM)))
    xf = x.reshape(_ROWS_TOTAL)
    enc = _make_enc().reshape(_SEQ_LEN * _EMBED_DIM)
    out = _sc_gather_add(wp, xf, enc)
    return out.reshape(_BATCH, _SEQ_LEN, _EMBED_DIM)
